# baseline (speedup 1.0000x reference)
import jax
import jax.numpy as jnp
from jax import lax
from jax.experimental import pallas as pl
from jax.experimental.pallas import tpu as pltpu
from jax.experimental.pallas import tpu_sc as plsc

KE = 332.0637

_NC = 2
_NS = 16
_NW = _NC * _NS
_L = 16

_E = 6400000
_T = _E // _NW
_C = 4000
_NCHUNK = _T // _C
_HALF = _C // 2
_VPH = _HALF // _L

_N_TAB = 100000
_H = _N_TAB // 2
_PC = 2000
_PCHUNK = _H // _PC
_PVPC = _PC // _L


def _body(d_hbm, a_hbm, b_hbm, iu_hbm, iv_hbm, out_hbm,
          ta_v, tb_v, u0, u1, v0, v1, w0, w1, o0, o1,
          insem0, insem1, outsem0, outsem1):
    sid = lax.axis_index("s")
    wid = sid * _NC + lax.axis_index("c")
    base = wid * _T

    u_v = (u0, u1)
    v_v = (v0, v1)
    w_v = (w0, w1)
    o_v = (o0, o1)
    insem = (insem0, insem1)
    outsem = (outsem0, outsem1)

    def start_in(c, b):
        off = base + c * _C
        pltpu.async_copy(iu_hbm.at[pl.ds(off, _C)], u_v[b], insem[b])
        pltpu.async_copy(iv_hbm.at[pl.ds(off, _C)], v_v[b], insem[b])
        pltpu.async_copy(d_hbm.at[pl.ds(off, _C)], w_v[b], insem[b])

    start_in(0, 0)

    core = lax.axis_index("c")
    xbase = core * _T
    stg_lo = w1.at[pl.ds(0, _PC)]
    stg_hi = w1.at[pl.ds(_PC, _PC)]

    def pack_slice(t_hbm, tab_base, c, cond):
        def do():
            off = c * _PC
            pltpu.async_copy(t_hbm.at[pl.ds(off, _PC)], stg_lo, outsem0)
            pltpu.async_copy(t_hbm.at[pl.ds(_H + off, _PC)], stg_hi,
                             outsem0)
            pltpu.make_async_copy(t_hbm.at[pl.ds(0, _PC)], stg_lo,
                                  outsem0).wait()
            pltpu.make_async_copy(t_hbm.at[pl.ds(0, _PC)], stg_hi,
                                  outsem0).wait()

            @plsc.parallel_loop(0, _PVPC, 1, unroll=5)
            def _(k):
                s = k * _L
                wl = plsc.bitcast(stg_lo[pl.ds(s, _L)], jnp.int32) + 0x8000
                wh = plsc.bitcast(stg_hi[pl.ds(s, _L)], jnp.int32) + 0x8000
                word = ((wl >> 16) & 0xFFFF) | (wh & -65536)
                o0[pl.ds(s, _L)] = plsc.bitcast(word, jnp.float32)

            pltpu.sync_copy(
                o0, out_hbm.at[pl.ds(xbase + tab_base + off, _PC)])

        if cond is None:
            do()
        else:
            pl.when(cond)(do)

    with jax.named_scope("pack_tables"):
        for i in range(2):
            c = sid + _NS * i
            cond = (c < _PCHUNK) if i else None
            pack_slice(a_hbm, 0, c, cond)
        plsc.subcore_barrier()
        pltpu.async_copy(out_hbm.at[pl.ds(xbase, _H)], ta_v, insem1)
        for i in range(2):
            c = sid + _NS * i
            cond = (c < _PCHUNK) if i else None
            pack_slice(b_hbm, _H, c, cond)
        plsc.subcore_barrier()
        pltpu.async_copy(out_hbm.at[pl.ds(xbase + _H, _H)], tb_v, outsem1)
        pltpu.make_async_copy(out_hbm.at[pl.ds(xbase, _H)], ta_v,
                              insem1).wait()
        pltpu.make_async_copy(out_hbm.at[pl.ds(xbase + _H, _H)], tb_v,
                              outsem1).wait()
        plsc.subcore_barrier()

    def wait_in(b):
        pltpu.make_async_copy(d_hbm.at[pl.ds(0, 3 * _C)],
                              ta_v.at[pl.ds(0, 3 * _C)], insem[b]).wait()

    def wait_out(h):
        pltpu.make_async_copy(o_v[h], out_hbm.at[pl.ds(0, _HALF)],
                              outsem[h]).wait()

    def pair_body(p, carry):
        for b in range(2):
            c = 2 * p + b
            if b == 0:
                start_in(c + 1, 1)
            else:
                @pl.when(p < _NCHUNK // 2 - 1)
                def _():
                    start_in(c + 1, 0)
            wait_in(b)

            ub, vb, wb = u_v[b], v_v[b], w_v[b]
            for h in range(2):
                if b == 0:
                    @pl.when(p > 0)
                    def _():
                        wait_out(h)
                else:
                    wait_out(h)
                oh = o_v[h]
                hoff = h * _HALF

                @plsc.parallel_loop(0, _VPH, 1, unroll=5)
                def _(k):
                    s = hoff + k * _L
                    iu = ub[pl.ds(s, _L)]
                    iv = vb[pl.ds(s, _L)]
                    gu = iu >= _H
                    gv = iv >= _H
                    ju = jnp.where(gu, iu - _H, iu)
                    jv = jnp.where(gv, iv - _H, iv)
                    wu = plsc.bitcast(plsc.load_gather(ta_v, [ju]),
                                      jnp.int32)
                    wv = plsc.bitcast(plsc.load_gather(tb_v, [jv]),
                                      jnp.int32)
                    qu = plsc.bitcast(
                        jnp.where(gu, wu & -65536, wu << 16), jnp.float32)
                    qv = plsc.bitcast(
                        jnp.where(gv, wv & -65536, wv << 16), jnp.float32)
                    dd = wb[pl.ds(s, _L)]
                    oh[pl.ds(k * _L, _L)] = (KE * qu) * qv / dd

                off = base + c * _C + hoff
                pltpu.async_copy(o_v[h], out_hbm.at[pl.ds(off, _HALF)],
                                 outsem[h])
        return carry

    with jax.named_scope("edge_stream"):
        lax.fori_loop(0, _NCHUNK // 2, pair_body, 0)
        wait_out(0)
        wait_out(1)


def kernel(mlmm_distances_uv, atomic_charges, mlmm_atomic_charges,
           mlmm_idx_u, mlmm_idx_v):
    mesh = plsc.VectorSubcoreMesh(core_axis_name="c", subcore_axis_name="s")
    run = pl.kernel(
        _body,
        out_type=jax.ShapeDtypeStruct((_E,), jnp.float32),
        mesh=mesh,
        compiler_params=pltpu.CompilerParams(needs_layout_passes=False),
        scratch_types=[
            pltpu.VMEM((_H,), jnp.float32),
            pltpu.VMEM((_H,), jnp.float32),
            pltpu.VMEM((_C,), jnp.int32),
            pltpu.VMEM((_C,), jnp.int32),
            pltpu.VMEM((_C,), jnp.int32),
            pltpu.VMEM((_C,), jnp.int32),
            pltpu.VMEM((_C,), jnp.float32),
            pltpu.VMEM((_C,), jnp.float32),
            pltpu.VMEM((_HALF,), jnp.float32),
            pltpu.VMEM((_HALF,), jnp.float32),
            pltpu.SemaphoreType.DMA,
            pltpu.SemaphoreType.DMA,
            pltpu.SemaphoreType.DMA,
            pltpu.SemaphoreType.DMA,
        ],
    )
    return run(mlmm_distances_uv, atomic_charges, mlmm_atomic_charges,
               mlmm_idx_u, mlmm_idx_v)


# speedup vs baseline: 1.0148x; 1.0148x over previous
import jax
import jax.numpy as jnp
from jax import lax
from jax.experimental import pallas as pl
from jax.experimental.pallas import tpu as pltpu
from jax.experimental.pallas import tpu_sc as plsc

KE = 332.0637

_NC = 2
_NS = 16
_NW = _NC * _NS
_L = 16

_E = 6400000
_T = _E // _NW
_C = 4000
_NCHUNK = _T // _C
_HALF = _C // 2
_VPH = _HALF // _L

_N_TAB = 100000
_H = _N_TAB // 2
_PC = 2000
_PCHUNK = _H // _PC
_PVPC = _PC // _L


def _body(d_hbm, a_hbm, b_hbm, iu_hbm, iv_hbm, out_hbm,
          ta_v, tb_v, u0, u1, v0, v1, w0, w1, o0, o1,
          insem0, insem1, outsem0, outsem1):
    sid = lax.axis_index("s")
    wid = sid * _NC + lax.axis_index("c")
    base = wid * _T

    u_v = (u0, u1)
    v_v = (v0, v1)
    w_v = (w0, w1)
    o_v = (o0, o1)
    insem = (insem0, insem1)
    outsem = (outsem0, outsem1)

    def start_in(c, b):
        off = base + c * _C
        pltpu.async_copy(iu_hbm.at[pl.ds(off, _C)], u_v[b], insem[b])
        pltpu.async_copy(iv_hbm.at[pl.ds(off, _C)], v_v[b], insem[b])
        pltpu.async_copy(d_hbm.at[pl.ds(off, _C)], w_v[b], insem[b])

    start_in(0, 0)

    core = lax.axis_index("c")
    xbase = core * _T
    stg_lo = w1.at[pl.ds(0, _PC)]
    stg_hi = w1.at[pl.ds(_PC, _PC)]

    def pack_slice(t_hbm, tab_base, c, cond):
        def do():
            off = c * _PC
            pltpu.async_copy(t_hbm.at[pl.ds(off, _PC)], stg_lo, outsem0)
            pltpu.async_copy(t_hbm.at[pl.ds(_H + off, _PC)], stg_hi,
                             outsem0)
            pltpu.make_async_copy(t_hbm.at[pl.ds(0, _PC)], stg_lo,
                                  outsem0).wait()
            pltpu.make_async_copy(t_hbm.at[pl.ds(0, _PC)], stg_hi,
                                  outsem0).wait()

            @plsc.parallel_loop(0, _PVPC, 1, unroll=5)
            def _(k):
                s = k * _L
                wl = plsc.bitcast(stg_lo[pl.ds(s, _L)], jnp.int32) + 0x8000
                wh = plsc.bitcast(stg_hi[pl.ds(s, _L)], jnp.int32) + 0x8000
                word = ((wl >> 16) & 0xFFFF) | (wh & -65536)
                o0[pl.ds(s, _L)] = plsc.bitcast(word, jnp.float32)

            pltpu.sync_copy(
                o0, out_hbm.at[pl.ds(xbase + tab_base + off, _PC)])

        if cond is None:
            do()
        else:
            pl.when(cond)(do)

    with jax.named_scope("pack_tables"):
        for i in range(2):
            c = sid + _NS * i
            cond = (c < _PCHUNK) if i else None
            pack_slice(a_hbm, 0, c, cond)
            pack_slice(b_hbm, _H, c, cond)
        plsc.subcore_barrier()
        pltpu.async_copy(out_hbm.at[pl.ds(xbase, _H)], ta_v, outsem0)
        pltpu.async_copy(out_hbm.at[pl.ds(xbase + _H, _H)], tb_v, outsem1)
        pltpu.make_async_copy(out_hbm.at[pl.ds(xbase, _H)], ta_v,
                              outsem0).wait()
        pltpu.make_async_copy(out_hbm.at[pl.ds(xbase + _H, _H)], tb_v,
                              outsem1).wait()
        plsc.subcore_barrier()

    def wait_in(b):
        pltpu.make_async_copy(iu_hbm.at[pl.ds(0, _C)], u_v[b],
                              insem[b]).wait()
        pltpu.make_async_copy(iv_hbm.at[pl.ds(0, _C)], v_v[b],
                              insem[b]).wait()
        pltpu.make_async_copy(d_hbm.at[pl.ds(0, _C)], w_v[b],
                              insem[b]).wait()

    def wait_out(h):
        pltpu.make_async_copy(o_v[h], out_hbm.at[pl.ds(0, _HALF)],
                              outsem[h]).wait()

    def pair_body(p, carry):
        for b in range(2):
            c = 2 * p + b
            if b == 0:
                start_in(c + 1, 1)
            else:
                @pl.when(p < _NCHUNK // 2 - 1)
                def _():
                    start_in(c + 1, 0)
            wait_in(b)

            ub, vb, wb = u_v[b], v_v[b], w_v[b]
            for h in range(2):
                if b == 0:
                    @pl.when(p > 0)
                    def _():
                        wait_out(h)
                else:
                    wait_out(h)
                oh = o_v[h]
                hoff = h * _HALF

                @plsc.parallel_loop(0, _VPH, 1, unroll=5)
                def _(k):
                    s = hoff + k * _L
                    iu = ub[pl.ds(s, _L)]
                    iv = vb[pl.ds(s, _L)]
                    gu = iu >= _H
                    gv = iv >= _H
                    ju = jnp.where(gu, iu - _H, iu)
                    jv = jnp.where(gv, iv - _H, iv)
                    wu = plsc.bitcast(plsc.load_gather(ta_v, [ju]),
                                      jnp.int32)
                    wv = plsc.bitcast(plsc.load_gather(tb_v, [jv]),
                                      jnp.int32)
                    qu = plsc.bitcast(
                        jnp.where(gu, wu & -65536, wu << 16), jnp.float32)
                    qv = plsc.bitcast(
                        jnp.where(gv, wv & -65536, wv << 16), jnp.float32)
                    dd = wb[pl.ds(s, _L)]
                    oh[pl.ds(k * _L, _L)] = (KE * qu) * qv / dd

                off = base + c * _C + hoff
                pltpu.async_copy(o_v[h], out_hbm.at[pl.ds(off, _HALF)],
                                 outsem[h])
        return carry

    with jax.named_scope("edge_stream"):
        lax.fori_loop(0, _NCHUNK // 2, pair_body, 0)
        wait_out(0)
        wait_out(1)


def kernel(mlmm_distances_uv, atomic_charges, mlmm_atomic_charges,
           mlmm_idx_u, mlmm_idx_v):
    mesh = plsc.VectorSubcoreMesh(core_axis_name="c", subcore_axis_name="s")
    run = pl.kernel(
        _body,
        out_type=jax.ShapeDtypeStruct((_E,), jnp.float32),
        mesh=mesh,
        compiler_params=pltpu.CompilerParams(needs_layout_passes=False),
        scratch_types=[
            pltpu.VMEM((_H,), jnp.float32),
            pltpu.VMEM((_H,), jnp.float32),
            pltpu.VMEM((_C,), jnp.int32),
            pltpu.VMEM((_C,), jnp.int32),
            pltpu.VMEM((_C,), jnp.int32),
            pltpu.VMEM((_C,), jnp.int32),
            pltpu.VMEM((_C,), jnp.float32),
            pltpu.VMEM((_C,), jnp.float32),
            pltpu.VMEM((_HALF,), jnp.float32),
            pltpu.VMEM((_HALF,), jnp.float32),
            pltpu.SemaphoreType.DMA,
            pltpu.SemaphoreType.DMA,
            pltpu.SemaphoreType.DMA,
            pltpu.SemaphoreType.DMA,
        ],
    )
    return run(mlmm_distances_uv, atomic_charges, mlmm_atomic_charges,
               mlmm_idx_u, mlmm_idx_v)
